# baseline (speedup 1.0000x reference)
import functools

import jax
import jax.numpy as jnp
from jax import lax
from jax.experimental import pallas as pl
from jax.experimental.pallas import tpu as pltpu
from jax.experimental.pallas import tpu_sc as plsc

VOCAB = 1_000_000
D = 64
B_ROWS = 4096
B_COLS = 200
B = B_ROWS * B_COLS

_NC = 2
_NS = 16
_NW = _NC * _NS
_B_PER_W = B // _NW
_XR_PER_W = B_ROWS // _NW
_P = 2
_CROWS = _P * _XR_PER_W
_NCHUNK = B_COLS // _P
_OUT_ELEMS = B * D
_BIS_IDX = False
_BIS_T = False


@functools.partial(
    pl.kernel,
    out_type=jax.ShapeDtypeStruct((B_COLS, 8, _NW, 8 * _XR_PER_W), jnp.float32),
    mesh=plsc.VectorSubcoreMesh(core_axis_name="c", subcore_axis_name="s"),
    scratch_types=[
        pltpu.VMEM((_B_PER_W,), jnp.int32),
        pltpu.VMEM((_CROWS,), jnp.int32),
        pltpu.VMEM((_CROWS,), jnp.int32),
        pltpu.VMEM((_CROWS, D), jnp.float32),
        pltpu.VMEM((_CROWS, D), jnp.float32),
        pltpu.VMEM((_P, 8, 8 * _XR_PER_W), jnp.float32),
        pltpu.VMEM((_P, 8, 8 * _XR_PER_W), jnp.float32),
        pltpu.SemaphoreType.DMA,
        pltpu.SemaphoreType.DMA,
        pltpu.SemaphoreType.DMA,
        pltpu.SemaphoreType.DMA,
    ],
    compiler_params=pltpu.CompilerParams(use_tc_tiling_on_sc=False, needs_layout_passes=False),
)
def _sc_gather(idx_hbm, table_hbm, out_hbm, idx_v, ic0, ic1, rows0, rows1,
               tb0, tb1, gsem0, gsem1, osem0, osem1):
    wid = lax.axis_index("s") * _NC + lax.axis_index("c")
    base = wid * _B_PER_W
    idx_c = (ic0, ic1)
    rows = (rows0, rows1)
    tb = (tb0, tb1)
    gsem = (gsem0, gsem1)
    osem = (osem0, osem1)

    iota = lax.iota(jnp.int32, 16)
    iota200 = iota * B_COLS

    pltpu.sync_copy(idx_hbm.at[pl.ds(base, _B_PER_W)], idx_v)

    def build_idx(i, b, stub=False):
        p0 = i * _P
        for p_local in range(_P):
            for r0 in range(0, _XR_PER_W, 16):
                if stub:
                    idx_c[b][pl.ds(p_local * _XR_PER_W + r0, 16)] = iota
                else:
                    addr = iota200 + (r0 * B_COLS + p0 + p_local)
                    v = plsc.load_gather(idx_v, [addr])
                    idx_c[b][pl.ds(p_local * _XR_PER_W + r0, 16)] = v

    def gather_copy(i, b):
        return pltpu.make_async_copy(
            table_hbm.at[idx_c[b]], rows[b], gsem[b])

    def transpose(b):
        for p_local in range(_P):
            row_vecs = [p_local * _XR_PER_W + r0 + iota
                        for r0 in range(0, _XR_PER_W, 16)]

            @plsc.parallel_loop(0, 8, 1, unroll=2)
            def dtbody(dt):
                for ds in range(8):
                    col = iota * 0 + (dt * 8 + ds)
                    for j, rv in enumerate(row_vecs):
                        v = plsc.load_gather(rows[b], [rv, col])
                        tb[b][p_local, dt, pl.ds(ds * _XR_PER_W + j * 16, 16)] = v

    def out_copies(i, b):
        p0 = i * _P
        cps = []
        for p_local in range(_P):
            cps.append(pltpu.make_async_copy(
                tb[b].at[p_local],
                out_hbm.at[p0 + p_local, :, wid, :], osem[b]))
        return cps

    build_idx(0, 0)
    gather_copy(0, 0).start()

    def chunk(i, b):
        @pl.when(i + 1 < _NCHUNK)
        def _():
            build_idx(i + 1, 1 - b)

        gather_copy(i, b).wait()

        @pl.when(i + 1 < _NCHUNK)
        def _():
            gather_copy(i + 1, 1 - b).start()

        @pl.when(i >= 2)
        def _():
            for cp in out_copies(i - 2, b):
                cp.wait()

        transpose(b)
        for cp in out_copies(i, b):
            cp.start()

    def outer(g, carry):
        chunk(2 * g, 0)
        chunk(2 * g + 1, 1)
        return carry

    lax.fori_loop(0, _NCHUNK // 2, outer, 0)
    for cp in out_copies(_NCHUNK - 2, 0):
        cp.wait()
    for cp in out_copies(_NCHUNK - 1, 1):
        cp.wait()


def kernel(x, table):
    flat = _sc_gather(x.reshape(B), table)
    out5 = flat.reshape(B_COLS, 8, _NW, 8, _XR_PER_W)
    return out5.transpose(2, 4, 0, 1, 3).reshape(B_ROWS, B_COLS, D)


# speedup vs baseline: 1.8404x; 1.8404x over previous
import functools

import jax
import jax.numpy as jnp
from jax import lax
from jax.experimental import pallas as pl
from jax.experimental.pallas import tpu as pltpu
from jax.experimental.pallas import tpu_sc as plsc

VOCAB = 1_000_000
D = 64
B_ROWS = 4096
B_COLS = 200
B = B_ROWS * B_COLS

_NC = 2
_NS = 16
_NW = _NC * _NS
_B_PER_W = B // _NW
_XR_PER_W = B_ROWS // _NW
_P = 2
_CROWS = _P * _XR_PER_W
_NCHUNK = B_COLS // _P
_OUT_ELEMS = B * D
_BIS_IDX = False
_BIS_T = False


@functools.partial(
    pl.kernel,
    out_type=jax.ShapeDtypeStruct((B_COLS, 8, _NW, 8, _XR_PER_W), jnp.float32),
    mesh=plsc.VectorSubcoreMesh(core_axis_name="c", subcore_axis_name="s"),
    scratch_types=[
        pltpu.VMEM((_B_PER_W,), jnp.int32),
        pltpu.VMEM((_CROWS,), jnp.int32),
        pltpu.VMEM((_CROWS,), jnp.int32),
        pltpu.VMEM((_CROWS, D), jnp.float32),
        pltpu.VMEM((_CROWS, D), jnp.float32),
        pltpu.VMEM((_P, D, _XR_PER_W + 1), jnp.float32),
        pltpu.VMEM((_P, D, _XR_PER_W + 1), jnp.float32),
        pltpu.SemaphoreType.DMA,
        pltpu.SemaphoreType.DMA,
        pltpu.SemaphoreType.DMA,
        pltpu.SemaphoreType.DMA,
    ],
    compiler_params=pltpu.CompilerParams(use_tc_tiling_on_sc=False, needs_layout_passes=False),
)
def _sc_gather(idx_hbm, table_hbm, out_hbm, idx_v, ic0, ic1, rows0, rows1,
               tb0, tb1, gsem0, gsem1, osem0, osem1):
    wid = lax.axis_index("s") * _NC + lax.axis_index("c")
    base = wid * _B_PER_W
    idx_c = (ic0, ic1)
    rows = (rows0, rows1)
    tb = (tb0, tb1)
    gsem = (gsem0, gsem1)
    osem = (osem0, osem1)

    iota = lax.iota(jnp.int32, 16)
    iota200 = iota * B_COLS

    pltpu.sync_copy(idx_hbm.at[pl.ds(base, _B_PER_W)], idx_v)

    def build_idx(i, b, stub=False):
        p0 = i * _P
        for p_local in range(_P):
            for r0 in range(0, _XR_PER_W, 16):
                if stub:
                    idx_c[b][pl.ds(p_local * _XR_PER_W + r0, 16)] = iota
                else:
                    addr = iota200 + (r0 * B_COLS + p0 + p_local)
                    v = plsc.load_gather(idx_v, [addr])
                    idx_c[b][pl.ds(p_local * _XR_PER_W + r0, 16)] = v

    def gather_copy(i, b):
        return pltpu.make_async_copy(
            table_hbm.at[idx_c[b]], rows[b], gsem[b])

    def transpose(b):
        for p_local in range(_P):
            for d0 in range(0, D, 16):
                dvec = iota + d0

                @plsc.parallel_loop(0, _XR_PER_W, 1, unroll=8)
                def rbody(r):
                    v = rows[b][p_local * _XR_PER_W + r, pl.ds(d0, 16)]
                    plsc.store_scatter(
                        tb[b], [dvec * 0 + p_local, dvec, dvec * 0 + r], v)

    def out_copies(i, b):
        p0 = i * _P
        cps = []
        for p_local in range(_P):
            for dt in range(8):
                cps.append(pltpu.make_async_copy(
                    tb[b].at[p_local, pl.ds(dt * 8, 8), pl.ds(0, _XR_PER_W)],
                    out_hbm.at[p0 + p_local, dt, wid, :, :], osem[b]))
        return cps

    build_idx(0, 0)
    gather_copy(0, 0).start()

    def chunk(i, b):
        @pl.when(i + 1 < _NCHUNK)
        def _():
            build_idx(i + 1, 1 - b)

        gather_copy(i, b).wait()

        @pl.when(i + 1 < _NCHUNK)
        def _():
            gather_copy(i + 1, 1 - b).start()

        @pl.when(i >= 2)
        def _():
            for cp in out_copies(i - 2, b):
                cp.wait()

        transpose(b)
        for cp in out_copies(i, b):
            cp.start()

    def outer(g, carry):
        chunk(2 * g, 0)
        chunk(2 * g + 1, 1)
        return carry

    lax.fori_loop(0, _NCHUNK // 2, outer, 0)
    for cp in out_copies(_NCHUNK - 2, 0):
        cp.wait()
    for cp in out_copies(_NCHUNK - 1, 1):
        cp.wait()


def kernel(x, table):
    flat = _sc_gather(x.reshape(B), table)
    out5 = flat.reshape(B_COLS, 8, _NW, 8, _XR_PER_W)
    return out5.transpose(2, 4, 0, 1, 3).reshape(B_ROWS, B_COLS, D)


# speedup vs baseline: 1.8419x; 1.0008x over previous
import functools

import jax
import jax.numpy as jnp
from jax import lax
from jax.experimental import pallas as pl
from jax.experimental.pallas import tpu as pltpu
from jax.experimental.pallas import tpu_sc as plsc

VOCAB = 1_000_000
D = 64
B_ROWS = 4096
B_COLS = 200
B = B_ROWS * B_COLS

_NC = 2
_NS = 16
_NW = _NC * _NS
_B_PER_W = B // _NW
_XR_PER_W = B_ROWS // _NW
_P = 2
_CROWS = _P * _XR_PER_W
_NCHUNK = B_COLS // _P
_OUT_ELEMS = B * D
_BIS_IDX = False
_BIS_T = False


@functools.partial(
    pl.kernel,
    out_type=jax.ShapeDtypeStruct((B_COLS, 8, _NW, 8, _XR_PER_W), jnp.float32),
    mesh=plsc.VectorSubcoreMesh(core_axis_name="c", subcore_axis_name="s"),
    scratch_types=[
        pltpu.VMEM((_B_PER_W,), jnp.int32),
        pltpu.VMEM((_CROWS,), jnp.int32),
        pltpu.VMEM((_CROWS,), jnp.int32),
        pltpu.VMEM((_CROWS, D), jnp.float32),
        pltpu.VMEM((_CROWS, D), jnp.float32),
        pltpu.VMEM((_P, D, _XR_PER_W + 1), jnp.float32),
        pltpu.VMEM((_P, D, _XR_PER_W + 1), jnp.float32),
        pltpu.SemaphoreType.DMA,
        pltpu.SemaphoreType.DMA,
        pltpu.SemaphoreType.DMA,
        pltpu.SemaphoreType.DMA,
    ],
    compiler_params=pltpu.CompilerParams(use_tc_tiling_on_sc=False, needs_layout_passes=False),
)
def _sc_gather(idx_hbm, table_hbm, out_hbm, idx_v, ic0, ic1, rows0, rows1,
               tb0, tb1, gsem0, gsem1, osem0, osem1):
    wid = lax.axis_index("s") * _NC + lax.axis_index("c")
    base = wid * _B_PER_W
    idx_c = (ic0, ic1)
    rows = (rows0, rows1)
    tb = (tb0, tb1)
    gsem = (gsem0, gsem1)
    osem = (osem0, osem1)

    iota = lax.iota(jnp.int32, 16)
    iota200 = iota * B_COLS

    pltpu.sync_copy(idx_hbm.at[pl.ds(base, _B_PER_W)], idx_v)

    def build_idx(i, b, stub=False):
        p0 = i * _P
        for p_local in range(_P):
            for r0 in range(0, _XR_PER_W, 16):
                if stub:
                    idx_c[b][pl.ds(p_local * _XR_PER_W + r0, 16)] = iota
                else:
                    addr = iota200 + (r0 * B_COLS + p0 + p_local)
                    v = plsc.load_gather(idx_v, [addr])
                    idx_c[b][pl.ds(p_local * _XR_PER_W + r0, 16)] = v

    def gather_copy(i, b):
        return pltpu.make_async_copy(
            table_hbm.at[idx_c[b]], rows[b], gsem[b])

    def transpose(b):
        for p_local in range(_P):
            for d0 in range(0, D, 16):
                dvec = iota + d0

                @plsc.parallel_loop(0, _XR_PER_W, 1, unroll=8)
                def rbody(r):
                    v = rows[b][p_local * _XR_PER_W + r, pl.ds(d0, 16)]
                    plsc.store_scatter(
                        tb[b], [dvec * 0 + p_local, dvec, dvec * 0 + r], v)

    def out_copies(i, b):
        p0 = i * _P
        cps = []
        for p_local in range(_P):
            for dt in range(8):
                cps.append(pltpu.make_async_copy(
                    tb[b].at[p_local, pl.ds(dt * 8, 8), pl.ds(0, _XR_PER_W)],
                    out_hbm.at[p0 + p_local, dt, wid, :, :], osem[b]))
        return cps

    build_idx(0, 0)
    gather_copy(0, 0).start()

    def chunk(i, b):
        @pl.when(i + 1 < _NCHUNK)
        def _():
            build_idx(i + 1, 1 - b)
            gather_copy(i + 1, 1 - b).start()

        gather_copy(i, b).wait()

        @pl.when(i >= 2)
        def _():
            for cp in out_copies(i - 2, b):
                cp.wait()

        transpose(b)
        for cp in out_copies(i, b):
            cp.start()

    def outer(g, carry):
        chunk(2 * g, 0)
        chunk(2 * g + 1, 1)
        return carry

    lax.fori_loop(0, _NCHUNK // 2, outer, 0)
    for cp in out_copies(_NCHUNK - 2, 0):
        cp.wait()
    for cp in out_copies(_NCHUNK - 1, 1):
        cp.wait()


def kernel(x, table):
    flat = _sc_gather(x.reshape(B), table)
    out5 = flat.reshape(B_COLS, 8, _NW, 8, _XR_PER_W)
    return out5.transpose(2, 4, 0, 1, 3).reshape(B_ROWS, B_COLS, D)
